# baseline (device time: 261196 ns/iter reference)
import functools

import jax
import jax.numpy as jnp
from jax import lax
from jax.experimental import pallas as pl
from jax.experimental.pallas import tpu as pltpu

N_DEV = 4
SQ = 2048
SKV_L = 2048
HQ = 8
DH = 128
DM = 1024
SCALE = 0.08838834764831843
NEG = -1e9
Q_TILE = 256
N_QT = SQ // Q_TILE

SR = 128
S_ROWS = 2 * SR
S_BASES = (0, SQ - SR)
S_LANES = DM + DH
M_OFF = DM
L_OFF = DM + HQ


def _qproj_body(x_ref, wq_ref, q_ref):
    q_ref[0] = jnp.dot(x_ref[...], wq_ref[...],
                       preferred_element_type=jnp.float32)


def _qproj(x2, Wq):
    return pl.pallas_call(
        _qproj_body,
        grid=(HQ,),
        in_specs=[
            pl.BlockSpec(memory_space=pltpu.VMEM),
            pl.BlockSpec((DM, DH), lambda h: (0, h)),
        ],
        out_specs=pl.BlockSpec((1, SQ, DH), lambda h: (h, 0, 0)),
        out_shape=jax.ShapeDtypeStruct((HQ, SQ, DH), jnp.float32),
        compiler_params=pltpu.CompilerParams(
            vmem_limit_bytes=60 * 1024 * 1024,
        ),
    )(x2, Wq)


def _flash_piece(q, k, v, qi_base, koff, rows):
    s = lax.dot_general(
        q, k, (((1,), (1,)), ((), ())), preferred_element_type=jnp.float32
    ) * SCALE
    qi = qi_base + lax.broadcasted_iota(jnp.int32, (rows, SKV_L), 0)
    ki = koff + lax.broadcasted_iota(jnp.int32, (rows, SKV_L), 1)
    mask = (jnp.abs(qi - ki) <= 128) | (ki < 32) | (qi < 32)
    s = jnp.where(mask, s, NEG)
    m = jnp.max(s, axis=1, keepdims=True)
    w = jnp.exp(s - m)
    lsum = jnp.sum(w, axis=1, keepdims=True)
    o = jnp.dot(w, v, preferred_element_type=jnp.float32)
    return o, m, lsum


def _mega_body(q_ref, k_ref, v_ref, out_ref,
               small_comm, run_st, k_buf, v_buf,
               ksem, vsem,
               bsend, brecv, stsend, strecv, rsend, rrecv):
    my = lax.axis_index("i")
    right = lax.rem(my + 1, N_DEV)
    left = lax.rem(my + N_DEV - 1, N_DEV)
    koff = my * SKV_L

    barrier = pltpu.get_barrier_semaphore()
    for nbr in (left, right):
        pl.semaphore_signal(
            barrier, inc=1, device_id=(nbr,),
            device_id_type=pl.DeviceIdType.MESH,
        )
    pl.semaphore_wait(barrier, 2)

    def kv_fetch(h, slot):
        return (
            pltpu.make_async_copy(k_ref.at[h], k_buf.at[slot], ksem.at[slot]),
            pltpu.make_async_copy(v_ref.at[h], v_buf.at[slot], vsem.at[slot]),
        )

    def ring_desc(hop):
        return pltpu.make_async_remote_copy(
            src_ref=small_comm.at[hop], dst_ref=small_comm.at[hop + 1],
            send_sem=rsend.at[hop], recv_sem=rrecv.at[hop + 1],
            device_id=(right,), device_id_type=pl.DeviceIdType.MESH,
        )

    def head_send(h, dest, slot):
        return pltpu.make_async_remote_copy(
            src_ref=out_ref.at[h], dst_ref=out_ref.at[h],
            send_sem=bsend.at[slot, h], recv_sem=brecv.at[h],
            device_id=(dest,), device_id_type=pl.DeviceIdType.MESH,
        )

    def stats_send(dest, slot):
        return pltpu.make_async_remote_copy(
            src_ref=run_st, dst_ref=run_st,
            send_sem=stsend.at[slot], recv_sem=strecv.at[0],
            device_id=(dest,), device_id_type=pl.DeviceIdType.MESH,
        )

    @pl.when(my != 0)
    def _():
        for c in kv_fetch(0, 0):
            c.start()
        for h in range(HQ):
            slot = h % 2
            for c in kv_fetch(h, slot):
                c.wait()
            if h < HQ - 1:
                for c in kv_fetch(h + 1, 1 - slot):
                    c.start()
            sl = slice(h * DH, (h + 1) * DH)
            for rb, qi_base in enumerate(S_BASES):
                base = rb * SR
                o, m, lsum = _flash_piece(
                    q_ref[h, qi_base:qi_base + SR, :],
                    k_buf[slot], v_buf[slot], qi_base, koff, SR,
                )
                small_comm[0, base:base + SR, sl] = o
                small_comm[0, base:base + SR, M_OFF + h:M_OFF + h + 1] = m
                small_comm[0, base:base + SR, L_OFF + h:L_OFF + h + 1] = lsum

    ring_desc(0).start()

    @pl.when(my == 0)
    def _():
        for c in kv_fetch(0, 0):
            c.start()

    for h in range(HQ):
        @pl.when(my == 0)
        def _(h=h):
            slot = h % 2
            for c in kv_fetch(h, slot):
                c.wait()
            if h < HQ - 1:
                for c in kv_fetch(h + 1, 1 - slot):
                    c.start()
            o, m, lsum = _flash_piece(
                q_ref[h, 0:Q_TILE, :], k_buf[slot], v_buf[slot],
                0, koff, Q_TILE,
            )
            out_ref[h, 0:Q_TILE, :] = o
            run_st[0:Q_TILE, h:h + 1] = m
            run_st[0:Q_TILE, HQ + h:HQ + h + 1] = lsum
            for qt in range(1, N_QT):
                r0 = qt * Q_TILE
                wlo = r0 - 128
                whi = min(r0 + Q_TILE + 128, SKV_L)
                width = whi - wlo
                q = q_ref[h, r0:r0 + Q_TILE, :]
                sg = lax.dot_general(
                    q, k_buf[slot][0:128], (((1,), (1,)), ((), ())),
                    preferred_element_type=jnp.float32,
                ) * SCALE
                sw = lax.dot_general(
                    q, k_buf[slot][wlo:whi], (((1,), (1,)), ((), ())),
                    preferred_element_type=jnp.float32,
                ) * SCALE
                qi_g = r0 + lax.broadcasted_iota(jnp.int32, (Q_TILE, 128), 0)
                ki_g = koff + lax.broadcasted_iota(jnp.int32, (Q_TILE, 128), 1)
                sg = jnp.where(ki_g < 32, sg, NEG)
                qi_w = r0 + lax.broadcasted_iota(
                    jnp.int32, (Q_TILE, width), 0)
                ki_w = koff + wlo + lax.broadcasted_iota(
                    jnp.int32, (Q_TILE, width), 1)
                sw = jnp.where(jnp.abs(qi_w - ki_w) <= 128, sw, NEG)
                m = jnp.maximum(
                    jnp.max(sg, axis=1, keepdims=True),
                    jnp.max(sw, axis=1, keepdims=True),
                )
                wg = jnp.exp(sg - m)
                ww = jnp.exp(sw - m)
                lsum = (jnp.sum(wg, axis=1, keepdims=True)
                        + jnp.sum(ww, axis=1, keepdims=True))
                o = (jnp.dot(wg, v_buf[slot][0:128],
                             preferred_element_type=jnp.float32)
                     + jnp.dot(ww, v_buf[slot][wlo:whi],
                               preferred_element_type=jnp.float32))
                out_ref[h, r0:r0 + Q_TILE, :] = o
                run_st[r0:r0 + Q_TILE, h:h + 1] = m
                run_st[r0:r0 + Q_TILE, HQ + h:HQ + h + 1] = lsum
            head_send(h, 1 if h < 4 else 3, 0).start()

        if h == 2:
            ring_desc(0).wait()
            ring_desc(1).start()
        elif h == 4:
            ring_desc(1).wait()
            ring_desc(2).start()
        elif h == 6:
            ring_desc(2).wait()

    @pl.when(my == 0)
    def _():
        stats_send(1, 0).start()
        stats_send(3, 1).start()
        for h in range(HQ):
            head_send(h, 1 if h < 4 else 3, 0).wait_send()
        stats_send(1, 0).wait_send()
        stats_send(3, 1).wait_send()

    @pl.when(my == 1)
    def _():
        for h in range(4):
            head_send(h, 0, 0).wait_recv()
            head_send(h, 2, 0).start()
        for h in range(4, HQ):
            head_send(h, 0, 0).wait_recv()
        stats_send(0, 0).wait_recv()
        stats_send(2, 0).start()
        for h in range(4):
            head_send(h, 2, 0).wait_send()
        stats_send(2, 0).wait_send()

    @pl.when(my == 3)
    def _():
        for h in range(4, HQ):
            head_send(h, 0, 0).wait_recv()
            head_send(h, 2, 0).start()
        for h in range(4):
            head_send(h, 0, 0).wait_recv()
        stats_send(0, 0).wait_recv()
        for h in range(4, HQ):
            head_send(h, 2, 0).wait_send()

    @pl.when(my == 2)
    def _():
        for h in range(4):
            head_send(h, 0, 0).wait_recv()
            head_send(h, 3, 0).start()
        for h in range(4, HQ):
            head_send(h, 0, 0).wait_recv()
            head_send(h, 1, 0).start()
        stats_send(0, 0).wait_recv()
        for h in range(4):
            head_send(h, 3, 0).wait_send()
        for h in range(4, HQ):
            head_send(h, 1, 0).wait_send()

    def merge_slot(s):
        for rb, r0 in enumerate(S_BASES):
            base = rb * SR
            m_old = run_st[r0:r0 + SR, 0:HQ]
            l_old = run_st[r0:r0 + SR, HQ:2 * HQ]
            m_in = small_comm[s, base:base + SR, M_OFF:M_OFF + HQ]
            l_in = small_comm[s, base:base + SR, L_OFF:L_OFF + HQ]
            m_new = jnp.maximum(m_old, m_in)
            a = jnp.exp(m_old - m_new)
            b = jnp.exp(m_in - m_new)
            run_st[r0:r0 + SR, 0:HQ] = m_new
            run_st[r0:r0 + SR, HQ:2 * HQ] = l_old * a + l_in * b
            for h in range(HQ):
                sl = slice(h * DH, (h + 1) * DH)
                out_ref[h, r0:r0 + SR, :] = (
                    out_ref[h, r0:r0 + SR, :] * a[:, h:h + 1]
                    + small_comm[s, base:base + SR, sl] * b[:, h:h + 1]
                )

    for s in range(1, N_DEV):
        origin = lax.rem(my - s + N_DEV, N_DEV)

        @pl.when(origin != 0)
        def _(s=s):
            merge_slot(s)

    @pl.when(my != 0)
    def _():
        merge_slot(0)

    for h in range(HQ):
        out_ref[h] = out_ref[h] / run_st[:, HQ + h:HQ + h + 1]

    @functools.partial(pl.run_scoped, second_barrier=pltpu.SemaphoreType.REGULAR)
    def _(second_barrier):
        for nbr in (left, right):
            pl.semaphore_signal(
                second_barrier, inc=1, device_id=(nbr,),
                device_id_type=pl.DeviceIdType.MESH,
            )
        pl.semaphore_wait(second_barrier, 2)


def _mega(Q_hm, Kt, Vt):
    return pl.pallas_call(
        _mega_body,
        in_specs=[
            pl.BlockSpec(memory_space=pltpu.VMEM),
            pl.BlockSpec(memory_space=pl.ANY),
            pl.BlockSpec(memory_space=pl.ANY),
        ],
        out_specs=pl.BlockSpec(memory_space=pltpu.VMEM),
        out_shape=jax.ShapeDtypeStruct((HQ, SQ, DH), jnp.float32),
        scratch_shapes=[
            pltpu.VMEM((N_DEV, S_ROWS, S_LANES), jnp.float32),
            pltpu.VMEM((SQ, 2 * HQ), jnp.float32),
            pltpu.VMEM((2, SKV_L, DH), jnp.float32),
            pltpu.VMEM((2, SKV_L, DH), jnp.float32),
            pltpu.SemaphoreType.DMA((2,)),
            pltpu.SemaphoreType.DMA((2,)),
            pltpu.SemaphoreType.DMA((2, HQ)),
            pltpu.SemaphoreType.DMA((HQ,)),
            pltpu.SemaphoreType.DMA((2,)),
            pltpu.SemaphoreType.DMA((1,)),
            pltpu.SemaphoreType.DMA((N_DEV - 1,)),
            pltpu.SemaphoreType.DMA((N_DEV,)),
        ],
        compiler_params=pltpu.CompilerParams(
            collective_id=0,
            vmem_limit_bytes=60 * 1024 * 1024,
        ),
    )(Q_hm, Kt, Vt)


def _proj_body(ctx_ref, wo_ref, out_ref):
    acc = jnp.dot(ctx_ref[0], wo_ref[0:DH, :],
                  preferred_element_type=jnp.float32)
    for h in range(1, HQ):
        acc = acc + jnp.dot(ctx_ref[h], wo_ref[h * DH:(h + 1) * DH, :],
                            preferred_element_type=jnp.float32)
    out_ref[...] = acc


def _out_proj(ctx_hm, Wo):
    return pl.pallas_call(
        _proj_body,
        in_specs=[pl.BlockSpec(memory_space=pltpu.VMEM)] * 2,
        out_specs=pl.BlockSpec(memory_space=pltpu.VMEM),
        out_shape=jax.ShapeDtypeStruct((SQ, DM), jnp.float32),
        compiler_params=pltpu.CompilerParams(
            vmem_limit_bytes=60 * 1024 * 1024,
        ),
    )(ctx_hm, Wo)


def kernel(x, Wq, K_ext, V_ext, Wo):
    x2 = x.reshape(SQ, DM)
    Kt = K_ext.reshape(SKV_L, HQ, DH).transpose(1, 0, 2)
    Vt = V_ext.reshape(SKV_L, HQ, DH).transpose(1, 0, 2)
    Q_hm = _qproj(x2, Wq)
    ctx_hm = _mega(Q_hm, Kt, Vt)
    out2 = _out_proj(ctx_hm, Wo)
    return out2.reshape(1, SQ, DM)


# device time: 179976 ns/iter; 1.4513x vs baseline; 1.4513x over previous
import functools

import jax
import jax.numpy as jnp
from jax import lax
from jax.experimental import pallas as pl
from jax.experimental.pallas import tpu as pltpu

N_DEV = 4
SQ = 2048
SKV_L = 2048
HQ = 8
DH = 128
DM = 1024
SCALE = 0.08838834764831843
NEG = -1e9
Q_TILE = 256
N_QT = SQ // Q_TILE

SR = 128
S_ROWS = 2 * SR
S_BASES = (0, SQ - SR)
S_LANES = DM + DH
M_OFF = DM
L_OFF = DM + HQ


def _qproj_body(x_ref, wq_ref, q_ref):
    q_ref[0] = jnp.dot(x_ref[...], wq_ref[...],
                       preferred_element_type=jnp.float32)


def _qproj(x2, Wq):
    return pl.pallas_call(
        _qproj_body,
        grid=(HQ,),
        in_specs=[
            pl.BlockSpec(memory_space=pltpu.VMEM),
            pl.BlockSpec((DM, DH), lambda h: (0, h)),
        ],
        out_specs=pl.BlockSpec((1, SQ, DH), lambda h: (h, 0, 0)),
        out_shape=jax.ShapeDtypeStruct((HQ, SQ, DH), jnp.float32),
        compiler_params=pltpu.CompilerParams(
            vmem_limit_bytes=60 * 1024 * 1024,
        ),
    )(x2, Wq)


def _flash_piece(q, k, v, qi_base, koff, rows):
    s = lax.dot_general(
        q, k, (((1,), (1,)), ((), ())), preferred_element_type=jnp.float32
    ) * SCALE
    qi = qi_base + lax.broadcasted_iota(jnp.int32, (rows, SKV_L), 0)
    ki = koff + lax.broadcasted_iota(jnp.int32, (rows, SKV_L), 1)
    mask = (jnp.abs(qi - ki) <= 128) | (ki < 32) | (qi < 32)
    s = jnp.where(mask, s, NEG)
    m = jnp.max(s, axis=1, keepdims=True)
    w = jnp.exp(s - m)
    lsum = jnp.sum(w, axis=1, keepdims=True)
    o = jnp.dot(w, v, preferred_element_type=jnp.float32)
    return o, m, lsum


def _mega_body(q_ref, k_ref, v_ref, out_ref,
               small_comm, run_st, k_buf, v_buf,
               ksem, vsem,
               bsend, brecv, stsend, strecv, rsend, rrecv):
    my = lax.axis_index("i")
    right = lax.rem(my + 1, N_DEV)
    left = lax.rem(my + N_DEV - 1, N_DEV)
    koff = my * SKV_L

    barrier = pltpu.get_barrier_semaphore()
    for nbr in (left, right):
        pl.semaphore_signal(
            barrier, inc=1, device_id=(nbr,),
            device_id_type=pl.DeviceIdType.MESH,
        )
    pl.semaphore_wait(barrier, 2)

    def kv_fetch(h, slot):
        return (
            pltpu.make_async_copy(k_ref.at[h], k_buf.at[slot], ksem.at[slot]),
            pltpu.make_async_copy(v_ref.at[h], v_buf.at[slot], vsem.at[slot]),
        )

    def ring_desc(hop):
        return pltpu.make_async_remote_copy(
            src_ref=small_comm.at[hop], dst_ref=small_comm.at[hop + 1],
            send_sem=rsend.at[hop], recv_sem=rrecv.at[hop + 1],
            device_id=(right,), device_id_type=pl.DeviceIdType.MESH,
        )

    def head_send(h, dest, slot):
        return pltpu.make_async_remote_copy(
            src_ref=out_ref.at[h], dst_ref=out_ref.at[h],
            send_sem=bsend.at[slot, h], recv_sem=brecv.at[h],
            device_id=(dest,), device_id_type=pl.DeviceIdType.MESH,
        )

    def stats_send(dest, slot):
        return pltpu.make_async_remote_copy(
            src_ref=run_st, dst_ref=run_st,
            send_sem=stsend.at[slot], recv_sem=strecv.at[0],
            device_id=(dest,), device_id_type=pl.DeviceIdType.MESH,
        )

    @pl.when(my != 0)
    def _():
        for c in kv_fetch(0, 0):
            c.start()
        for h in range(HQ):
            slot = h % 2
            for c in kv_fetch(h, slot):
                c.wait()
            if h < HQ - 1:
                for c in kv_fetch(h + 1, 1 - slot):
                    c.start()
            sl = slice(h * DH, (h + 1) * DH)
            for rb, qi_base in enumerate(S_BASES):
                base = rb * SR
                o, m, lsum = _flash_piece(
                    q_ref[h, qi_base:qi_base + SR, :],
                    k_buf[slot], v_buf[slot], qi_base, koff, SR,
                )
                small_comm[0, base:base + SR, sl] = o
                small_comm[0, base:base + SR, M_OFF + h:M_OFF + h + 1] = m
                small_comm[0, base:base + SR, L_OFF + h:L_OFF + h + 1] = lsum

    ring_desc(0).start()

    @pl.when(my == 0)
    def _():
        for c in kv_fetch(0, 0):
            c.start()

    for h in range(HQ):
        @pl.when(my == 0)
        def _(h=h):
            slot = h % 2
            for c in kv_fetch(h, slot):
                c.wait()
            if h < HQ - 1:
                for c in kv_fetch(h + 1, 1 - slot):
                    c.start()
            o, m, lsum = _flash_piece(
                q_ref[h, 0:Q_TILE, :], k_buf[slot], v_buf[slot],
                0, koff, Q_TILE,
            )
            out_ref[h, 0:Q_TILE, :] = o.astype(jnp.bfloat16)
            run_st[0:Q_TILE, h:h + 1] = m
            run_st[0:Q_TILE, HQ + h:HQ + h + 1] = lsum
            for qt in range(1, N_QT):
                r0 = qt * Q_TILE
                wlo = r0 - 128
                whi = min(r0 + Q_TILE + 128, SKV_L)
                width = whi - wlo
                q = q_ref[h, r0:r0 + Q_TILE, :]
                sg = lax.dot_general(
                    q, k_buf[slot][0:128], (((1,), (1,)), ((), ())),
                    preferred_element_type=jnp.float32,
                ) * SCALE
                sw = lax.dot_general(
                    q, k_buf[slot][wlo:whi], (((1,), (1,)), ((), ())),
                    preferred_element_type=jnp.float32,
                ) * SCALE
                qi_g = r0 + lax.broadcasted_iota(jnp.int32, (Q_TILE, 128), 0)
                ki_g = koff + lax.broadcasted_iota(jnp.int32, (Q_TILE, 128), 1)
                sg = jnp.where(ki_g < 32, sg, NEG)
                qi_w = r0 + lax.broadcasted_iota(
                    jnp.int32, (Q_TILE, width), 0)
                ki_w = koff + wlo + lax.broadcasted_iota(
                    jnp.int32, (Q_TILE, width), 1)
                sw = jnp.where(jnp.abs(qi_w - ki_w) <= 128, sw, NEG)
                m = jnp.maximum(
                    jnp.max(sg, axis=1, keepdims=True),
                    jnp.max(sw, axis=1, keepdims=True),
                )
                wg = jnp.exp(sg - m)
                ww = jnp.exp(sw - m)
                lsum = (jnp.sum(wg, axis=1, keepdims=True)
                        + jnp.sum(ww, axis=1, keepdims=True))
                o = (jnp.dot(wg, v_buf[slot][0:128],
                             preferred_element_type=jnp.float32)
                     + jnp.dot(ww, v_buf[slot][wlo:whi],
                               preferred_element_type=jnp.float32))
                out_ref[h, r0:r0 + Q_TILE, :] = o.astype(jnp.bfloat16)
                run_st[r0:r0 + Q_TILE, h:h + 1] = m
                run_st[r0:r0 + Q_TILE, HQ + h:HQ + h + 1] = lsum
            head_send(h, 1, 0).start()
            head_send(h, 3, 1).start()

        if h == 2:
            ring_desc(0).wait()
            ring_desc(1).start()
        elif h == 4:
            ring_desc(1).wait()
            ring_desc(2).start()
        elif h == 6:
            ring_desc(2).wait()

    @pl.when(my == 0)
    def _():
        stats_send(1, 0).start()
        stats_send(3, 1).start()
        for h in range(HQ):
            head_send(h, 1, 0).wait_send()
            head_send(h, 3, 1).wait_send()
        stats_send(1, 0).wait_send()
        stats_send(3, 1).wait_send()

    @pl.when(my == 1)
    def _():
        for h in range(4):
            head_send(h, 0, 0).wait_recv()
            head_send(h, 2, 0).start()
        for h in range(4, HQ):
            head_send(h, 0, 0).wait_recv()
        stats_send(0, 0).wait_recv()
        stats_send(2, 0).start()
        for h in range(4):
            head_send(h, 2, 0).wait_send()
        stats_send(2, 0).wait_send()

    @pl.when(my == 3)
    def _():
        for h in range(4, HQ):
            head_send(h, 0, 0).wait_recv()
            head_send(h, 2, 0).start()
        for h in range(4):
            head_send(h, 0, 0).wait_recv()
        stats_send(0, 0).wait_recv()
        for h in range(4, HQ):
            head_send(h, 2, 0).wait_send()

    @pl.when(my == 2)
    def _():
        for h in range(HQ):
            head_send(h, 0, 0).wait_recv()
        stats_send(0, 0).wait_recv()

    def merge_slot(s):
        for rb, r0 in enumerate(S_BASES):
            base = rb * SR
            m_old = run_st[r0:r0 + SR, 0:HQ]
            l_old = run_st[r0:r0 + SR, HQ:2 * HQ]
            m_in = small_comm[s, base:base + SR, M_OFF:M_OFF + HQ]
            l_in = small_comm[s, base:base + SR, L_OFF:L_OFF + HQ]
            m_new = jnp.maximum(m_old, m_in)
            a = jnp.exp(m_old - m_new)
            b = jnp.exp(m_in - m_new)
            run_st[r0:r0 + SR, 0:HQ] = m_new
            run_st[r0:r0 + SR, HQ:2 * HQ] = l_old * a + l_in * b
            for h in range(HQ):
                sl = slice(h * DH, (h + 1) * DH)
                out_ref[h, r0:r0 + SR, :] = (
                    out_ref[h, r0:r0 + SR, :].astype(jnp.float32)
                    * a[:, h:h + 1]
                    + small_comm[s, base:base + SR, sl] * b[:, h:h + 1]
                ).astype(jnp.bfloat16)

    for s in range(1, N_DEV):
        origin = lax.rem(my - s + N_DEV, N_DEV)

        @pl.when(origin != 0)
        def _(s=s):
            merge_slot(s)

    @pl.when(my != 0)
    def _():
        merge_slot(0)

    for h in range(HQ):
        out_ref[h] = (
            out_ref[h].astype(jnp.float32) / run_st[:, HQ + h:HQ + h + 1]
        ).astype(jnp.bfloat16)

    @functools.partial(pl.run_scoped, second_barrier=pltpu.SemaphoreType.REGULAR)
    def _(second_barrier):
        for nbr in (left, right):
            pl.semaphore_signal(
                second_barrier, inc=1, device_id=(nbr,),
                device_id_type=pl.DeviceIdType.MESH,
            )
        pl.semaphore_wait(second_barrier, 2)


def _mega(Q_hm, Kt, Vt):
    return pl.pallas_call(
        _mega_body,
        in_specs=[
            pl.BlockSpec(memory_space=pltpu.VMEM),
            pl.BlockSpec(memory_space=pl.ANY),
            pl.BlockSpec(memory_space=pl.ANY),
        ],
        out_specs=pl.BlockSpec(memory_space=pltpu.VMEM),
        out_shape=jax.ShapeDtypeStruct((HQ, SQ, DH), jnp.bfloat16),
        scratch_shapes=[
            pltpu.VMEM((N_DEV, S_ROWS, S_LANES), jnp.float32),
            pltpu.VMEM((SQ, 2 * HQ), jnp.float32),
            pltpu.VMEM((2, SKV_L, DH), jnp.float32),
            pltpu.VMEM((2, SKV_L, DH), jnp.float32),
            pltpu.SemaphoreType.DMA((2,)),
            pltpu.SemaphoreType.DMA((2,)),
            pltpu.SemaphoreType.DMA((2, HQ)),
            pltpu.SemaphoreType.DMA((HQ,)),
            pltpu.SemaphoreType.DMA((2,)),
            pltpu.SemaphoreType.DMA((1,)),
            pltpu.SemaphoreType.DMA((N_DEV - 1,)),
            pltpu.SemaphoreType.DMA((N_DEV,)),
        ],
        compiler_params=pltpu.CompilerParams(
            collective_id=0,
            vmem_limit_bytes=60 * 1024 * 1024,
        ),
    )(Q_hm, Kt, Vt)


def _proj_body(ctx_ref, wo_ref, out_ref):
    acc = jnp.dot(ctx_ref[0], wo_ref[0:DH, :],
                  preferred_element_type=jnp.float32)
    for h in range(1, HQ):
        acc = acc + jnp.dot(ctx_ref[h], wo_ref[h * DH:(h + 1) * DH, :],
                            preferred_element_type=jnp.float32)
    out_ref[...] = acc


def _out_proj(ctx_hm, Wo):
    return pl.pallas_call(
        _proj_body,
        in_specs=[pl.BlockSpec(memory_space=pltpu.VMEM)] * 2,
        out_specs=pl.BlockSpec(memory_space=pltpu.VMEM),
        out_shape=jax.ShapeDtypeStruct((SQ, DM), jnp.float32),
        compiler_params=pltpu.CompilerParams(
            vmem_limit_bytes=60 * 1024 * 1024,
        ),
    )(ctx_hm, Wo)


def kernel(x, Wq, K_ext, V_ext, Wo):
    x2 = x.reshape(SQ, DM)
    Kt = K_ext.reshape(SKV_L, HQ, DH).transpose(1, 0, 2)
    Vt = V_ext.reshape(SKV_L, HQ, DH).transpose(1, 0, 2)
    Q_hm = _qproj(x2, Wq)
    ctx_hm = _mega(Q_hm, Kt, Vt)
    out2 = _out_proj(ctx_hm, Wo.astype(jnp.bfloat16))
    return out2.reshape(1, SQ, DM)
